# Initial kernel scaffold; baseline (speedup 1.0000x reference)
import jax
import jax.numpy as jnp
from jax.experimental import pallas as pl


def kernel(scalar_feats, edge_feats, edge_index, W1, b1, W2, b2, W3, b3, W4, b4, g1, be1, g2, be2):
    raise NotImplementedError("write your pallas kernel here")


# speedup vs baseline: 3.3166x; 3.3166x over previous
import functools

import jax
import jax.numpy as jnp
from jax import lax
from jax.experimental import pallas as pl
from jax.experimental.pallas import tpu as pltpu
from jax.experimental.pallas import tpu_sc as plsc

N = 10000
E = 320000
D = 128
DE = 16

_NC = 2
_NS = 16
_NW = _NC * _NS
_BATCH = 128
_NBAT = E // _BATCH
_PW = _NBAT // _NW
_REM = _NBAT - _PW * _NW
_NB_G = 6
_ITERS_G = _PW // _NB_G
_NB_S = 2
_ITERS_S = _PW // _NB_S
_ROWS_N = N // _NS

def _silu(x):
    return x * jax.nn.sigmoid(x)


def _ln(x, g, b, eps=1e-5):
    mu = jnp.mean(x, axis=-1, keepdims=True)
    var = jnp.var(x, axis=-1, keepdims=True)
    return (x - mu) / jnp.sqrt(var + eps) * g + b


def _sc_gather_body(p_hbm, src_hbm, g_hbm, idx_v, rows_v, sem):
    cid = lax.axis_index("c")
    sid = lax.axis_index("s")
    wid = sid * _NC + cid

    def do_chunk(b0, nb):
        pltpu.sync_copy(src_hbm.at[pl.ds(b0, nb)], idx_v.at[pl.ds(0, nb)])
        descs = [
            pltpu.async_copy(
                p_hbm.at[idx_v.at[j]],
                rows_v.at[pl.ds(j * _BATCH, _BATCH)],
                sem,
            )
            for j in range(nb)
        ]
        for d in descs:
            d.wait()
        pltpu.sync_copy(
            rows_v.at[pl.ds(0, nb * _BATCH)],
            g_hbm.at[pl.ds(b0 * _BATCH, nb * _BATCH)],
        )

    def body(i, carry):
        do_chunk(wid * _PW + i * _NB_G, _NB_G)
        return carry

    lax.fori_loop(0, _ITERS_G, body, 0)

    @pl.when(wid < _REM)
    def _():
        do_chunk(_PW * _NW + wid, 1)


def _sc_scatter_body(m_hbm, dst_hbm, zero_hbm, out_hbm, idx_v, rows_v, acc_sh, sem):
    cid = lax.axis_index("c")
    sid = lax.axis_index("s")
    wid = sid * _NC + cid

    pltpu.sync_copy(
        zero_hbm.at[pl.ds(sid * _ROWS_N, _ROWS_N)],
        acc_sh.at[pl.ds(sid * _ROWS_N, _ROWS_N)],
    )
    plsc.subcore_barrier()

    def do_chunk(b0, nb):
        pltpu.sync_copy(dst_hbm.at[pl.ds(b0, nb)], idx_v.at[pl.ds(0, nb)])
        pltpu.sync_copy(
            m_hbm.at[pl.ds(b0 * _BATCH, nb * _BATCH)],
            rows_v.at[pl.ds(0, nb * _BATCH)],
        )
        for j in range(nb):
            pltpu.sync_copy(
                rows_v.at[pl.ds(j * _BATCH, _BATCH)],
                acc_sh.at[idx_v.at[j]],
                add=True,
            )

    def body(i, carry):
        do_chunk(wid * _PW + i * _NB_S, _NB_S)
        return carry

    lax.fori_loop(0, _ITERS_S, body, 0)

    @pl.when(wid < _REM)
    def _():
        do_chunk(_PW * _NW + wid, 1)

    plsc.subcore_barrier()
    pltpu.sync_copy(
        acc_sh.at[pl.ds(sid * _ROWS_N, _ROWS_N)],
        out_hbm.at[cid, pl.ds(sid * _ROWS_N, _ROWS_N)],
    )


@functools.lru_cache(maxsize=None)
def _sc_kernels():
    mesh = plsc.VectorSubcoreMesh(
        core_axis_name="c", subcore_axis_name="s",
        num_cores=_NC, num_subcores=_NS)
    params = pltpu.CompilerParams(use_tc_tiling_on_sc=False)
    gather = pl.kernel(
        _sc_gather_body,
        out_type=jax.ShapeDtypeStruct((E, D), jnp.float32),
        mesh=mesh,
        compiler_params=params,
        scratch_types=[
            pltpu.VMEM((_NB_G, _BATCH), jnp.int32),
            pltpu.VMEM((_NB_G * _BATCH, D), jnp.float32),
            pltpu.SemaphoreType.DMA,
        ],
    )
    scatter = pl.kernel(
        _sc_scatter_body,
        out_type=jax.ShapeDtypeStruct((_NC, N, D), jnp.float32),
        mesh=mesh,
        compiler_params=params,
        scratch_types=[
            pltpu.VMEM((_NB_S, _BATCH), jnp.int32),
            pltpu.VMEM((_NB_S * _BATCH, D), jnp.float32),
            pltpu.VMEM_SHARED((N, D), jnp.float32),
            pltpu.SemaphoreType.DMA,
        ],
    )
    return gather, scatter


def _p_body(x_ref, w1a_ref, p_ref):
    p_ref[...] = jnp.dot(x_ref[...], w1a_ref[...],
                         preferred_element_type=jnp.float32)


_BE = 3200


def _mlp_body(g_ref, ef_ref, w1b_ref, b1_ref, w2_ref, b2_ref, o_ref):
    q = jnp.dot(ef_ref[...], w1b_ref[...],
                preferred_element_type=jnp.float32) + b1_ref[...]
    m1 = _silu(g_ref[...] + q)
    o_ref[...] = _silu(jnp.dot(m1, w2_ref[...],
                               preferred_element_type=jnp.float32) + b2_ref[...])


def _final_body(x_ref, parts_ref, w3_ref, b3_ref, w4_ref, b4_ref,
                g1_ref, be1_ref, g2_ref, be2_ref, o_ref):
    agg = (parts_ref[0] + parts_ref[1]) * 0.1
    h = _ln(x_ref[...] + agg, g1_ref[...], be1_ref[...])
    r = _silu(jnp.dot(h, w3_ref[...],
                      preferred_element_type=jnp.float32) + b3_ref[...])
    r = _silu(jnp.dot(r, w4_ref[...],
                      preferred_element_type=jnp.float32) + b4_ref[...])
    o_ref[...] = _ln(h + r, g2_ref[...], be2_ref[...])


def kernel(scalar_feats, edge_feats, edge_index, W1, b1, W2, b2, W3, b3,
           W4, b4, g1, be1, g2, be2):
    src2d = edge_index[0].reshape(_NBAT, _BATCH)
    dst2d = edge_index[1].reshape(_NBAT, _BATCH)
    w1a = W1[:D]
    w1b = W1[D:]

    p = pl.pallas_call(
        _p_body,
        out_shape=jax.ShapeDtypeStruct((N, D), jnp.float32),
    )(scalar_feats, w1a)

    sc_gather, sc_scatter = _sc_kernels()
    g = sc_gather(p, src2d)

    m = pl.pallas_call(
        _mlp_body,
        grid=(E // _BE,),
        in_specs=[
            pl.BlockSpec((_BE, D), lambda i: (i, 0)),
            pl.BlockSpec((_BE, DE), lambda i: (i, 0)),
            pl.BlockSpec((DE, D), lambda i: (0, 0)),
            pl.BlockSpec((1, D), lambda i: (0, 0)),
            pl.BlockSpec((D, D), lambda i: (0, 0)),
            pl.BlockSpec((1, D), lambda i: (0, 0)),
        ],
        out_specs=pl.BlockSpec((_BE, D), lambda i: (i, 0)),
        out_shape=jax.ShapeDtypeStruct((E, D), jnp.float32),
    )(g, edge_feats, w1b, b1.reshape(1, D), W2, b2.reshape(1, D))

    zeros = jnp.zeros((N, D), jnp.float32)
    parts = sc_scatter(m, dst2d, zeros)

    out = pl.pallas_call(
        _final_body,
        out_shape=jax.ShapeDtypeStruct((N, D), jnp.float32),
    )(scalar_feats, parts, W3, b3.reshape(1, D), W4, b4.reshape(1, D),
      g1.reshape(1, D), be1.reshape(1, D), g2.reshape(1, D), be2.reshape(1, D))
    return out
